# baseline (speedup 1.0000x reference)
import jax
import jax.numpy as jnp
from jax.experimental import pallas as pl


ROWS_PER_BLOCK = 256


def _trim_block(seq_ref, pe_ref, out_ref):
    out_ref[...] = seq_ref[...] * 2.0 + pe_ref[...]


def kernel(seq, times, pe):
    del times
    b, t, d = seq.shape
    pe2 = pe[0]
    k = t // ROWS_PER_BLOCK

    out = pl.pallas_call(
        _trim_block,
        grid=(k, b),
        in_specs=[
            pl.BlockSpec((1, ROWS_PER_BLOCK, d), lambda ki, bi: (bi, ki, 0)),
            pl.BlockSpec((ROWS_PER_BLOCK, d), lambda ki, bi: (ki, 0)),
        ],
        out_specs=pl.BlockSpec((1, ROWS_PER_BLOCK, d), lambda ki, bi: (bi, ki, 0)),
        out_shape=jax.ShapeDtypeStruct((b, t, d), seq.dtype),
    )(seq, pe2)

    mask = jnp.ones((b, 1), dtype=bool)
    return (out[:, None, :, :], mask)


# speedup vs baseline: 2.1140x; 2.1140x over previous
import jax
import jax.numpy as jnp
from jax.experimental import pallas as pl


def _trim_block(seq_ref, pe_ref, out_ref):
    out_ref[0] = seq_ref[...] * 2.0 + pe_ref[...]


def kernel(seq, times, pe):
    del times
    b, t, d = seq.shape
    pe2 = pe[0]

    out = pl.pallas_call(
        _trim_block,
        grid=(b,),
        in_specs=[
            pl.BlockSpec((1, t, d), lambda bi: (bi, 0, 0)),
            pl.BlockSpec((t, d), lambda bi: (0, 0)),
        ],
        out_specs=pl.BlockSpec((1, 1, t, d), lambda bi: (bi, 0, 0, 0)),
        out_shape=jax.ShapeDtypeStruct((b, 1, t, d), seq.dtype),
    )(seq, pe2)

    mask = jnp.ones((b, 1), dtype=bool)
    return (out, mask)
